# baseline (speedup 1.0000x reference)
import functools

import jax
import jax.numpy as jnp
from jax import lax
from jax.experimental import pallas as pl
from jax.experimental.pallas import tpu as pltpu
from jax.experimental.pallas import tpu_sc as plsc

NUM_SEGMENTS = 16384
NUM_ATOM_TYPES = 100
TYPE_PAD = 128
HIDDEN = 64
N_ATOMS = 800000

NUM_CORES = 2
NUM_SUBCORES = 16
NW = NUM_CORES * NUM_SUBCORES

SEG_PER_TILE = NUM_SEGMENTS // NW
HIST_TILE = SEG_PER_TILE * TYPE_PAD

IBUF = 12800
N_PADDED = N_ATOMS + IBUF
NBND = 64
BND_STRIDE = 256
PAD_IDX = NUM_SEGMENTS * TYPE_PAD


def _sc_hist_body(z_hbm, b_hbm, bnd_hbm, out_hbm, zb0, zb1, bb0, bb1,
                  bndvec, hist, sem):
    c = lax.axis_index("c")
    s = lax.axis_index("s")
    wid = c * NUM_SUBCORES + s
    lo_seg = wid * SEG_PER_TILE
    hi_seg = lo_seg + SEG_PER_TILE

    pltpu.sync_copy(bnd_hbm, bndvec)
    lanes = jax.lax.broadcasted_iota(jnp.int32, (16,), 0)

    def _bnd(w):
        vec = bndvec[pl.ds((w // 16) * 16, 16)]
        onehot = jnp.int32(1) - jnp.minimum(jnp.abs(lanes - w % 16),
                                            jnp.int32(1))
        return jnp.sum(vec * onehot)

    b_lo = _bnd(wid)
    b_hi = _bnd(NW + wid)
    start0 = (b_lo // 8) * 8
    nbatch = (b_hi - start0 + IBUF - 1) // IBUF

    def _base(t):
        bu = start0 + t * IBUF
        return jnp.minimum(bu, N_ATOMS - IBUF), bu

    def _fire(t, zb, bb):
        base, _ = _base(t)
        pltpu.async_copy(z_hbm.at[pl.ds(base, IBUF)], zb, sem)
        pltpu.async_copy(b_hbm.at[pl.ds(base, IBUF)], bb, sem)

    @pl.when(nbatch > 0)
    def _fire0():
        _fire(0, zb0, bb0)

    zeros = jnp.zeros((16,), jnp.float32)

    @pl.loop(0, SEG_PER_TILE)
    def _zero(i):
        for j in range(TYPE_PAD // 16):
            hist[i, pl.ds(j * 16, 16)] = zeros

    one = jnp.float32(1.0)
    zero = jnp.float32(0.0)

    def _step(t, zcur, bcur, znxt, bnxt):
        base, bu = _base(t)
        off = bu - base
        pltpu.make_async_copy(z_hbm.at[pl.ds(base, IBUF)], zcur,
                              sem).wait()
        pltpu.make_async_copy(b_hbm.at[pl.ds(base, IBUF)], bcur,
                              sem).wait()

        @pl.when(t + 1 < nbatch)
        def _prefetch():
            _fire(t + 1, znxt, bnxt)

        @plsc.parallel_loop(0, IBUF // 16, unroll=4)
        def _scan(k):
            vz = zcur[pl.ds(k * 16, 16)]
            vb = bcur[pl.ds(k * 16, 16)]
            m = (vb >= lo_seg) & (vb < hi_seg) & (lanes >= off - k * 16)
            vseg = jnp.where(m, vb - lo_seg, 0)
            plsc.addupdate_scatter(hist, [vseg, vz],
                                   jnp.where(m, one, zero))

    @pl.loop(0, nbatch)
    def _batch(t):
        @pl.when(t % 2 == 0)
        def _even():
            _step(t, zb0, bb0, zb1, bb1)

        @pl.when(t % 2 == 1)
        def _odd():
            _step(t, zb1, bb1, zb0, bb0)

    pltpu.sync_copy(hist, out_hbm.at[pl.ds(wid * SEG_PER_TILE,
                                           SEG_PER_TILE)])


@functools.cache
def _sc_hist():
  return pl.kernel(
    _sc_hist_body,
    out_type=jax.ShapeDtypeStruct((NUM_SEGMENTS, TYPE_PAD), jnp.float32),
    mesh=plsc.VectorSubcoreMesh(core_axis_name="c", subcore_axis_name="s",
                                num_cores=NUM_CORES,
                                num_subcores=NUM_SUBCORES),
    compiler_params=pltpu.CompilerParams(needs_layout_passes=False),
    scratch_types=[
        pltpu.VMEM((IBUF,), jnp.int32),
        pltpu.VMEM((IBUF,), jnp.int32),
        pltpu.VMEM((IBUF,), jnp.int32),
        pltpu.VMEM((IBUF,), jnp.int32),
        pltpu.VMEM((NBND,), jnp.int32),
        pltpu.VMEM((SEG_PER_TILE, TYPE_PAD), jnp.float32),
        pltpu.SemaphoreType.DMA,
    ],
  )


SEG_BLK = 2048


def _mlp_body(h_ref, emb_ref, w1_ref, b1_ref, w2_ref, b2_ref, out_ref):
    hist = h_ref[...]
    counts = jnp.sum(hist, axis=1, keepdims=True)
    sums = jnp.dot(hist, emb_ref[...], preferred_element_type=jnp.float32)
    pooled = sums / jnp.maximum(counts, 1.0)
    h = jnp.maximum(
        jnp.dot(pooled, w1_ref[...], preferred_element_type=jnp.float32)
        + b1_ref[...], 0.0)
    out = (jnp.dot(h, w2_ref[...], preferred_element_type=jnp.float32)
           + b2_ref[...])
    out_ref[...] = out[:, 0]


def _mlp(h, emb, W1, b1, W2, b2):
    return pl.pallas_call(
        _mlp_body,
        grid=(NUM_SEGMENTS // SEG_BLK,),
        in_specs=[
            pl.BlockSpec((SEG_BLK, TYPE_PAD), lambda i: (i, 0)),
            pl.BlockSpec((TYPE_PAD, HIDDEN), lambda i: (0, 0)),
            pl.BlockSpec((HIDDEN, HIDDEN), lambda i: (0, 0)),
            pl.BlockSpec((1, HIDDEN), lambda i: (0, 0)),
            pl.BlockSpec((HIDDEN, 1), lambda i: (0, 0)),
            pl.BlockSpec((1, 1), lambda i: (0, 0)),
        ],
        out_specs=pl.BlockSpec((SEG_BLK,), lambda i: (i,)),
        out_shape=jax.ShapeDtypeStruct((NUM_SEGMENTS,), jnp.float32),
    )(h, emb, W1, b1, W2, b2)


@jax.jit
def kernel(z, batch, emb, W1, b1, W2, b2):
    z = z.astype(jnp.int32)
    batch = batch.astype(jnp.int32)
    sample = batch[::BND_STRIDE]
    pos = jnp.searchsorted(
        sample, jnp.arange(0, NUM_SEGMENTS + 1, SEG_PER_TILE,
                           dtype=jnp.int32),
        method="compare_all").astype(jnp.int32)
    lo = jnp.maximum(pos - 1, 0) * BND_STRIDE
    hi = jnp.minimum(pos * BND_STRIDE, N_ATOMS)
    bnd = jnp.concatenate([lo[:NW], hi[1:NW + 1]])

    hist = _sc_hist()(z, batch, bnd)
    embp = jnp.pad(emb, ((0, TYPE_PAD - NUM_ATOM_TYPES), (0, 0)))
    return _mlp(hist, embp, W1, b1.reshape(1, HIDDEN), W2,
                b2.reshape(1, 1))


# speedup vs baseline: 1.0198x; 1.0198x over previous
import functools

import jax
import jax.numpy as jnp
from jax import lax
from jax.experimental import pallas as pl
from jax.experimental.pallas import tpu as pltpu
from jax.experimental.pallas import tpu_sc as plsc

NUM_SEGMENTS = 16384
NUM_ATOM_TYPES = 100
TYPE_PAD = 128
HIDDEN = 64
N_ATOMS = 800000

NUM_CORES = 2
NUM_SUBCORES = 16
NW = NUM_CORES * NUM_SUBCORES

SEG_PER_TILE = NUM_SEGMENTS // NW
IBUF = 12800
NBND = 64
BND_STRIDE = 256


def _sc_hist_body(z_hbm, b_hbm, bnd_hbm, out_hbm, zb0, zb1, bb0, bb1,
                  bndvec, hist, sem):
    c = lax.axis_index("c")
    s = lax.axis_index("s")
    wid = c * NUM_SUBCORES + s
    lo_seg = wid * SEG_PER_TILE
    hi_seg = lo_seg + SEG_PER_TILE

    pltpu.sync_copy(bnd_hbm, bndvec)
    lanes = jax.lax.broadcasted_iota(jnp.int32, (16,), 0)

    def _bnd(w):
        vec = bndvec[pl.ds((w // 16) * 16, 16)]
        onehot = jnp.int32(1) - jnp.minimum(jnp.abs(lanes - w % 16),
                                            jnp.int32(1))
        return jnp.sum(vec * onehot)

    b_lo = _bnd(wid)
    b_hi = _bnd(NW + wid)
    start0 = (b_lo // 8) * 8
    nbatch = (b_hi - start0 + IBUF - 1) // IBUF

    def _base(t):
        bu = start0 + t * IBUF
        return jnp.minimum(bu, N_ATOMS - IBUF), bu

    def _fire(t, zb, bb):
        base, _ = _base(t)
        pltpu.async_copy(z_hbm.at[pl.ds(base, IBUF)], zb, sem)
        pltpu.async_copy(b_hbm.at[pl.ds(base, IBUF)], bb, sem)

    @pl.when(nbatch > 0)
    def _fire0():
        _fire(0, zb0, bb0)

    zeros = jnp.zeros((16,), jnp.float32)

    @pl.loop(0, SEG_PER_TILE)
    def _zero(i):
        for j in range(TYPE_PAD // 16):
            hist[i, pl.ds(j * 16, 16)] = zeros

    one = jnp.float32(1.0)
    zero = jnp.float32(0.0)

    def _step(t, zcur, bcur, znxt, bnxt):
        base, bu = _base(t)
        off = bu - base
        pltpu.make_async_copy(z_hbm.at[pl.ds(base, IBUF)], zcur,
                              sem).wait()
        pltpu.make_async_copy(b_hbm.at[pl.ds(base, IBUF)], bcur,
                              sem).wait()

        @pl.when(t + 1 < nbatch)
        def _prefetch():
            _fire(t + 1, znxt, bnxt)

        @plsc.parallel_loop(0, IBUF // 16, unroll=4)
        def _scan(k):
            vz = zcur[pl.ds(k * 16, 16)]
            vb = bcur[pl.ds(k * 16, 16)]
            m = (vb >= lo_seg) & (vb < hi_seg) & (lanes >= off - k * 16)
            vseg = jnp.where(m, vb - lo_seg, 0)
            plsc.addupdate_scatter(hist, [vseg, vz],
                                   jnp.where(m, one, zero))

    @pl.loop(0, nbatch)
    def _batch(t):
        @pl.when(t % 2 == 0)
        def _even():
            _step(t, zb0, bb0, zb1, bb1)

        @pl.when(t % 2 == 1)
        def _odd():
            _step(t, zb1, bb1, zb0, bb0)

    pltpu.sync_copy(hist, out_hbm.at[pl.ds(wid * SEG_PER_TILE,
                                           SEG_PER_TILE)])


@functools.cache
def _sc_hist():
  return pl.kernel(
    _sc_hist_body,
    out_type=jax.ShapeDtypeStruct((NUM_SEGMENTS, TYPE_PAD), jnp.float32),
    mesh=plsc.VectorSubcoreMesh(core_axis_name="c", subcore_axis_name="s",
                                num_cores=NUM_CORES,
                                num_subcores=NUM_SUBCORES),
    compiler_params=pltpu.CompilerParams(needs_layout_passes=False),
    scratch_types=[
        pltpu.VMEM((IBUF,), jnp.int32),
        pltpu.VMEM((IBUF,), jnp.int32),
        pltpu.VMEM((IBUF,), jnp.int32),
        pltpu.VMEM((IBUF,), jnp.int32),
        pltpu.VMEM((NBND,), jnp.int32),
        pltpu.VMEM((SEG_PER_TILE, TYPE_PAD), jnp.float32),
        pltpu.SemaphoreType.DMA,
    ],
  )


SEG_BLK = 4096


def _mlp_body(h_ref, emb_ref, w1_ref, b1_ref, w2_ref, b2_ref, out_ref):
    hist = h_ref[...]
    counts = jnp.sum(hist, axis=1, keepdims=True)
    sums = jnp.dot(hist, emb_ref[...], preferred_element_type=jnp.float32)
    pooled = sums / jnp.maximum(counts, 1.0)
    h = jnp.maximum(
        jnp.dot(pooled, w1_ref[...], preferred_element_type=jnp.float32)
        + b1_ref[...], 0.0)
    out = (jnp.dot(h, w2_ref[...], preferred_element_type=jnp.float32)
           + b2_ref[...])
    out_ref[...] = out[:, 0]


def _mlp(h, emb, W1, b1, W2, b2):
    return pl.pallas_call(
        _mlp_body,
        grid=(NUM_SEGMENTS // SEG_BLK,),
        in_specs=[
            pl.BlockSpec((SEG_BLK, TYPE_PAD), lambda i: (i, 0)),
            pl.BlockSpec((TYPE_PAD, HIDDEN), lambda i: (0, 0)),
            pl.BlockSpec((HIDDEN, HIDDEN), lambda i: (0, 0)),
            pl.BlockSpec((1, HIDDEN), lambda i: (0, 0)),
            pl.BlockSpec((HIDDEN, 1), lambda i: (0, 0)),
            pl.BlockSpec((1, 1), lambda i: (0, 0)),
        ],
        out_specs=pl.BlockSpec((SEG_BLK,), lambda i: (i,)),
        out_shape=jax.ShapeDtypeStruct((NUM_SEGMENTS,), jnp.float32),
    )(h, emb, W1, b1, W2, b2)


@jax.jit
def kernel(z, batch, emb, W1, b1, W2, b2):
    z = z.astype(jnp.int32)
    batch = batch.astype(jnp.int32)
    sample = batch[::BND_STRIDE]
    pos = jnp.searchsorted(
        sample, jnp.arange(0, NUM_SEGMENTS + 1, SEG_PER_TILE,
                           dtype=jnp.int32),
        method="compare_all").astype(jnp.int32)
    lo = jnp.maximum(pos - 1, 0) * BND_STRIDE
    hi = jnp.minimum(pos * BND_STRIDE, N_ATOMS)
    bnd = jnp.concatenate([lo[:NW], hi[1:NW + 1]])

    hist = _sc_hist()(z, batch, bnd)
    embp = jnp.pad(emb, ((0, TYPE_PAD - NUM_ATOM_TYPES), (0, 0)))
    return _mlp(hist, embp, W1, b1.reshape(1, HIDDEN), W2,
                b2.reshape(1, 1))
